# baseline (speedup 1.0000x reference)
import functools

import jax
import jax.numpy as jnp
from jax.experimental import pallas as pl

_K_MIN = 1.0
_K_MAX = 12.0
_ALPHA = 12.0
_TOPK = 12
_NEG = -1e30


_BPB = 2


def _body(x_ref, fc1_w_ref, fc1_b_ref, fc2_w_ref, fc2_b_ref,
          k1_w_ref, k1_b_ref, k2_w_ref, k2_b_ref,
          w1_w_ref, w1_b_ref, w2_w_ref, w2_b_ref, o_ref):
    for bi in range(_BPB):
        o_ref[bi] = _one_batch(
            x_ref[bi], fc1_w_ref, fc1_b_ref, fc2_w_ref, fc2_b_ref,
            k1_w_ref, k1_b_ref, k2_w_ref, k2_b_ref,
            w1_w_ref, w1_b_ref, w2_w_ref, w2_b_ref)


def _one_batch(xb, fc1_w_ref, fc1_b_ref, fc2_w_ref, fc2_b_ref,
               k1_w_ref, k1_b_ref, k2_w_ref, k2_b_ref,
               w1_w_ref, w1_b_ref, w2_w_ref, w2_b_ref):
    n = xb.shape[0]

    pooled = jnp.mean(xb, axis=0, keepdims=True)
    t = jnp.maximum(
        jnp.dot(pooled, k1_w_ref[...], preferred_element_type=jnp.float32)
        + k1_b_ref[...], 0.0)
    kl = jnp.dot(t, k2_w_ref[...], preferred_element_type=jnp.float32) + k2_b_ref[...]
    kc = _K_MIN + jax.nn.sigmoid(kl) * (_K_MAX - _K_MIN)
    t2 = jnp.maximum(
        jnp.dot(pooled, w1_w_ref[...], preferred_element_type=jnp.float32)
        + w1_b_ref[...], 0.0)
    wl = jnp.dot(t2, w2_w_ref[...], preferred_element_type=jnp.float32) + w2_b_ref[...]

    k_i = [kc[0, i] for i in range(3)]
    l_i = [wl[0, i] for i in range(3)]
    lmax = jnp.maximum(jnp.maximum(l_i[0], l_i[1]), l_i[2])
    e_i = [jnp.exp(l - lmax) for l in l_i]
    esum = e_i[0] + e_i[1] + e_i[2]
    w_i = [e / esum for e in e_i]

    h1 = jnp.maximum(
        jnp.dot(xb.astype(jnp.bfloat16), fc1_w_ref[...],
                preferred_element_type=jnp.float32)
        + fc1_b_ref[...], 0.0)
    h = jnp.dot(h1.astype(jnp.bfloat16), fc2_w_ref[...],
                preferred_element_type=jnp.float32) + fc2_b_ref[...]

    gram = jnp.dot(h, h.T, preferred_element_type=jnp.float32)
    rows = jax.lax.broadcasted_iota(jnp.int32, (n, n), 0)
    cols = jax.lax.broadcasted_iota(jnp.int32, (n, n), 1)
    eye = (rows == cols).astype(jnp.float32)
    sq_row = jnp.sum(gram * eye, axis=0, keepdims=True)
    logits = 2.0 * gram - sq_row

    m = [[jax.nn.sigmoid(_ALPHA * (k_i[i] - float(j + 1))) for j in range(_TOPK)]
         for i in range(3)]
    work = logits
    acc = [jnp.zeros((n, n), jnp.float32) for _ in range(3)]
    v0 = None
    ssum = None
    for j in range(_TOPK):
        cur = jnp.max(work, axis=1, keepdims=True)
        sel = work >= cur
        if j == 0:
            v0 = cur
            s = jnp.ones((n, 1), jnp.float32)
            ssum = s
        else:
            s = jnp.exp(cur - v0)
            ssum = ssum + s
        for i in range(3):
            acc[i] = acc[i] + jnp.where(sel, s * m[i][j], 0.0)
        work = jnp.where(sel, _NEG, work)

    attn = functools.reduce(jnp.add, [
        (w_i[i] / (jnp.sum(acc[i], axis=1, keepdims=True) + 1e-8 * ssum)) * acc[i]
        for i in range(3)])

    return jnp.dot(attn, h, preferred_element_type=jnp.float32)


def kernel(x, fc1_w, fc1_b, fc2_w, fc2_b, k1_w, k1_b, k2_w, k2_b,
           w1_w, w1_b, w2_w, w2_b):
    B, N, C = x.shape
    H = fc1_w.shape[1]
    k2_wp = jnp.pad(k2_w, ((0, 0), (0, 128 - k2_w.shape[1])))
    k2_bp = jnp.pad(k2_b, (0, 128 - k2_b.shape[0])).reshape(1, 128)
    w2_wp = jnp.pad(w2_w, ((0, 0), (0, 128 - w2_w.shape[1])))
    w2_bp = jnp.pad(w2_b, (0, 128 - w2_b.shape[0])).reshape(1, 128)

    const = lambda shape: pl.BlockSpec(shape, lambda b: (0,) * len(shape))
    return pl.pallas_call(
        _body,
        grid=(B // _BPB,),
        in_specs=[
            pl.BlockSpec((_BPB, N, C), lambda b: (b, 0, 0)),
            const((C, H)), const((1, H)),
            const((H, C)), const((1, C)),
            const((C, 128)), const((1, 128)),
            const((128, 128)), const((1, 128)),
            const((C, 128)), const((1, 128)),
            const((128, 128)), const((1, 128)),
        ],
        out_specs=pl.BlockSpec((_BPB, N, C), lambda b: (b, 0, 0)),
        out_shape=jax.ShapeDtypeStruct((B, N, C), jnp.float32),
    )(x, fc1_w.astype(jnp.bfloat16), fc1_b.reshape(1, H),
      fc2_w.astype(jnp.bfloat16), fc2_b.reshape(1, C),
      k1_w, k1_b.reshape(1, 128), k2_wp, k2_bp,
      w1_w, w1_b.reshape(1, 128), w2_wp, w2_bp)


# speedup vs baseline: 1.3982x; 1.3982x over previous
import functools

import jax
import jax.numpy as jnp
from jax.experimental import pallas as pl

_K_MIN = 1.0
_K_MAX = 12.0
_ALPHA = 12.0
_TOPK = 12
_NEG = -1e30


_BPB = 4


def _body(x_ref, fc1_w_ref, fc1_b_ref, fc2_w_ref, fc2_b_ref,
          k1_w_ref, k1_b_ref, k2_w_ref, k2_b_ref,
          w1_w_ref, w1_b_ref, w2_w_ref, w2_b_ref, o_ref):
    a = [None] * _BPB
    t = [None] * _BPB
    a[0] = _stage_mxu(x_ref[0], fc1_w_ref, fc1_b_ref, fc2_w_ref, fc2_b_ref,
                      k1_w_ref, k1_b_ref, k2_w_ref, k2_b_ref,
                      w1_w_ref, w1_b_ref, w2_w_ref, w2_b_ref)
    for bi in range(1, _BPB):
        a[bi] = _stage_mxu(x_ref[bi], fc1_w_ref, fc1_b_ref, fc2_w_ref,
                           fc2_b_ref, k1_w_ref, k1_b_ref, k2_w_ref, k2_b_ref,
                           w1_w_ref, w1_b_ref, w2_w_ref, w2_b_ref)
        t[bi - 1] = _stage_topk(a[bi - 1][1])
    t[_BPB - 1] = _stage_topk(a[_BPB - 1][1])
    for bi in range(_BPB):
        o_ref[bi] = _stage_final(a[bi], t[bi])


def _stage_mxu(xb, fc1_w_ref, fc1_b_ref, fc2_w_ref, fc2_b_ref,
               k1_w_ref, k1_b_ref, k2_w_ref, k2_b_ref,
               w1_w_ref, w1_b_ref, w2_w_ref, w2_b_ref):
    n = xb.shape[0]

    pooled = jnp.dot(jnp.full((1, n), 1.0 / n, jnp.float32), xb,
                     preferred_element_type=jnp.float32)
    t = jnp.maximum(
        jnp.dot(pooled, k1_w_ref[...], preferred_element_type=jnp.float32)
        + k1_b_ref[...], 0.0)
    kl = jnp.dot(t, k2_w_ref[...], preferred_element_type=jnp.float32) + k2_b_ref[...]
    kc = _K_MIN + jax.nn.sigmoid(kl) * (_K_MAX - _K_MIN)
    t2 = jnp.maximum(
        jnp.dot(pooled, w1_w_ref[...], preferred_element_type=jnp.float32)
        + w1_b_ref[...], 0.0)
    wl = jnp.dot(t2, w2_w_ref[...], preferred_element_type=jnp.float32) + w2_b_ref[...]

    k_i = [kc[0, i] for i in range(3)]
    l_i = [wl[0, i] for i in range(3)]
    lmax = jnp.maximum(jnp.maximum(l_i[0], l_i[1]), l_i[2])
    e_i = [jnp.exp(l - lmax) for l in l_i]
    esum = e_i[0] + e_i[1] + e_i[2]
    w_i = [e / esum for e in e_i]

    h1 = jnp.maximum(
        jnp.dot(xb, fc1_w_ref[...], preferred_element_type=jnp.float32)
        + fc1_b_ref[...], 0.0)
    h = jnp.dot(h1, fc2_w_ref[...], preferred_element_type=jnp.float32) + fc2_b_ref[...]

    gram = jnp.dot(h, h.T, preferred_element_type=jnp.float32)
    rows = jax.lax.broadcasted_iota(jnp.int32, (n, n), 0)
    cols = jax.lax.broadcasted_iota(jnp.int32, (n, n), 1)
    eye = (rows == cols).astype(jnp.float32)
    sq_row = jnp.sum(gram * eye, axis=0, keepdims=True)
    logits = 2.0 * gram - sq_row
    return (h, logits, k_i, w_i)


def _stage_topk(logits):
    n = logits.shape[0]
    work = logits
    rank = jnp.zeros((n, n), jnp.float32)
    v0 = None
    for j in range(_TOPK):
        cur = jnp.max(work, axis=1, keepdims=True)
        if j == 0:
            v0 = cur
        sel = work >= cur
        rank = jnp.where(sel, float(j + 1), rank)
        work = jnp.where(sel, _NEG, work)
    return (rank, v0)


def _stage_final(a, t):
    h, logits, k_i, w_i = a
    rank, v0 = t
    inside = rank > 0.0
    e = jnp.where(inside, jnp.exp(logits - v0), 0.0)
    esum = jnp.sum(e, axis=1, keepdims=True)
    attn = jnp.zeros(logits.shape, jnp.float32)
    for i in range(3):
        mi = jax.nn.sigmoid(_ALPHA * (k_i[i] - rank))
        num_i = e * mi
        den_i = jnp.sum(num_i, axis=1, keepdims=True) + 1e-8 * esum
        attn = attn + (w_i[i] / den_i) * num_i

    return jnp.dot(attn, h, preferred_element_type=jnp.float32)


def kernel(x, fc1_w, fc1_b, fc2_w, fc2_b, k1_w, k1_b, k2_w, k2_b,
           w1_w, w1_b, w2_w, w2_b):
    B, N, C = x.shape
    H = fc1_w.shape[1]
    k2_wp = jnp.pad(k2_w, ((0, 0), (0, 128 - k2_w.shape[1])))
    k2_bp = jnp.pad(k2_b, (0, 128 - k2_b.shape[0])).reshape(1, 128)
    w2_wp = jnp.pad(w2_w, ((0, 0), (0, 128 - w2_w.shape[1])))
    w2_bp = jnp.pad(w2_b, (0, 128 - w2_b.shape[0])).reshape(1, 128)

    const = lambda shape: pl.BlockSpec(shape, lambda b: (0,) * len(shape))
    return pl.pallas_call(
        _body,
        grid=(B // _BPB,),
        in_specs=[
            pl.BlockSpec((_BPB, N, C), lambda b: (b, 0, 0)),
            const((C, H)), const((1, H)),
            const((H, C)), const((1, C)),
            const((C, 128)), const((1, 128)),
            const((128, 128)), const((1, 128)),
            const((C, 128)), const((1, 128)),
            const((128, 128)), const((1, 128)),
        ],
        out_specs=pl.BlockSpec((_BPB, N, C), lambda b: (b, 0, 0)),
        out_shape=jax.ShapeDtypeStruct((B, N, C), jnp.float32),
    )(x, fc1_w, fc1_b.reshape(1, H), fc2_w, fc2_b.reshape(1, C),
      k1_w, k1_b.reshape(1, 128), k2_wp, k2_bp,
      w1_w, w1_b.reshape(1, 128), w2_wp, w2_bp)


# speedup vs baseline: 1.4071x; 1.0064x over previous
import jax
import jax.numpy as jnp
from jax.experimental import pallas as pl

_K_MIN = 1.0
_K_MAX = 12.0
_ALPHA = 12.0
_TOPK = 12
_SENT = -1.0e6

_BPB = 4


def _logits_stage(hb):
    n = hb.shape[0]
    gram = jnp.dot(hb, hb.T, preferred_element_type=jnp.float32)
    rows = jax.lax.broadcasted_iota(jnp.int32, (n, n), 0)
    cols = jax.lax.broadcasted_iota(jnp.int32, (n, n), 1)
    eye = (rows == cols).astype(jnp.float32)
    sq_row = jnp.sum(gram * eye, axis=0, keepdims=True)
    return 2.0 * gram - sq_row


def _topk_stage(logits):
    work = logits
    v0 = None
    for j in range(_TOPK):
        cur = jnp.max(work, axis=1, keepdims=True)
        if j == 0:
            v0 = cur
        sel = work >= cur
        work = jnp.where(sel, _SENT - float(j + 1), work)
    rank = jnp.where(work < _SENT + 0.5, _SENT - work, 0.0)
    return rank, v0


def _final_stage(hb, logits, rank, v0, k_i, w_i):
    e = jnp.where(rank > 0.0, jnp.exp(logits - v0), 0.0)
    esum = jnp.sum(e, axis=1, keepdims=True)
    attn = jnp.zeros(logits.shape, jnp.float32)
    for i in range(3):
        mi = jax.nn.sigmoid(_ALPHA * (k_i[i] - rank))
        num_i = e * mi
        den_i = jnp.sum(num_i, axis=1, keepdims=True) + 1e-8 * esum
        attn = attn + (w_i[i] / den_i) * num_i
    return jnp.dot(attn, hb, preferred_element_type=jnp.float32)


def _body(x_ref, fc1_w_ref, fc1_b_ref, fc2_w_ref, fc2_b_ref,
          k1_w_ref, k1_b_ref, k2_w_ref, k2_b_ref,
          w1_w_ref, w1_b_ref, w2_w_ref, w2_b_ref, o_ref):
    n, c = x_ref.shape[1], x_ref.shape[2]
    m = _BPB * n
    xs = x_ref[...].reshape(m, c)

    prow = jax.lax.broadcasted_iota(jnp.int32, (_BPB, m), 0)
    pcol = jax.lax.broadcasted_iota(jnp.int32, (_BPB, m), 1)
    selmat = jnp.where(prow == pcol // n, 1.0 / n, 0.0)
    pooled = jnp.dot(selmat, xs, preferred_element_type=jnp.float32)

    t = jnp.maximum(
        jnp.dot(pooled, k1_w_ref[...], preferred_element_type=jnp.float32)
        + k1_b_ref[...], 0.0)
    kl = jnp.dot(t, k2_w_ref[...], preferred_element_type=jnp.float32) + k2_b_ref[...]
    kc = _K_MIN + jax.nn.sigmoid(kl) * (_K_MAX - _K_MIN)
    t2 = jnp.maximum(
        jnp.dot(pooled, w1_w_ref[...], preferred_element_type=jnp.float32)
        + w1_b_ref[...], 0.0)
    wl = jnp.dot(t2, w2_w_ref[...], preferred_element_type=jnp.float32) + w2_b_ref[...]

    k_l, w_l = [], []
    for bi in range(_BPB):
        k_l.append([kc[bi, i] for i in range(3)])
        l_i = [wl[bi, i] for i in range(3)]
        lmax = jnp.maximum(jnp.maximum(l_i[0], l_i[1]), l_i[2])
        e_i = [jnp.exp(l - lmax) for l in l_i]
        esum = e_i[0] + e_i[1] + e_i[2]
        w_l.append([e / esum for e in e_i])

    h1 = jnp.maximum(
        jnp.dot(xs, fc1_w_ref[...], preferred_element_type=jnp.float32)
        + fc1_b_ref[...], 0.0)

    hb = [None] * _BPB
    logits = [None] * _BPB
    ranks = [None] * _BPB
    for s in range(_BPB + 2):
        if s < _BPB:
            hb[s] = jnp.dot(h1[s * n:(s + 1) * n], fc2_w_ref[...],
                            preferred_element_type=jnp.float32) + fc2_b_ref[...]
            logits[s] = _logits_stage(hb[s])
        if 1 <= s <= _BPB:
            ranks[s - 1] = _topk_stage(logits[s - 1])
        if s >= 2:
            bi = s - 2
            o_ref[bi] = _final_stage(hb[bi], logits[bi], ranks[bi][0],
                                     ranks[bi][1], k_l[bi], w_l[bi])


def kernel(x, fc1_w, fc1_b, fc2_w, fc2_b, k1_w, k1_b, k2_w, k2_b,
           w1_w, w1_b, w2_w, w2_b):
    B, N, C = x.shape
    H = fc1_w.shape[1]
    k2_wp = jnp.pad(k2_w, ((0, 0), (0, 128 - k2_w.shape[1])))
    k2_bp = jnp.pad(k2_b, (0, 128 - k2_b.shape[0])).reshape(1, 128)
    w2_wp = jnp.pad(w2_w, ((0, 0), (0, 128 - w2_w.shape[1])))
    w2_bp = jnp.pad(w2_b, (0, 128 - w2_b.shape[0])).reshape(1, 128)

    const = lambda shape: pl.BlockSpec(shape, lambda b: (0,) * len(shape))
    return pl.pallas_call(
        _body,
        grid=(B // _BPB,),
        in_specs=[
            pl.BlockSpec((_BPB, N, C), lambda b: (b, 0, 0)),
            const((C, H)), const((1, H)),
            const((H, C)), const((1, C)),
            const((C, 128)), const((1, 128)),
            const((128, 128)), const((1, 128)),
            const((C, 128)), const((1, 128)),
            const((128, 128)), const((1, 128)),
        ],
        out_specs=pl.BlockSpec((_BPB, N, C), lambda b: (b, 0, 0)),
        out_shape=jax.ShapeDtypeStruct((B, N, C), jnp.float32),
    )(x, fc1_w, fc1_b.reshape(1, H), fc2_w, fc2_b.reshape(1, C),
      k1_w, k1_b.reshape(1, 128), k2_wp, k2_bp,
      w1_w, w1_b.reshape(1, 128), w2_wp, w2_bp)
